# Initial kernel scaffold; baseline (speedup 1.0000x reference)
import functools

import jax
import jax.numpy as jnp
from jax import lax
from jax.experimental import pallas as pl
from jax.experimental.pallas import tpu as pltpu
from jax.experimental.pallas import tpu_sc as plsc

VOCAB = 1000000
DIM = 64
BATCH = 16384
HIST = 50

NC = 2
NS = 16
NW = NC * NS
ROWS_PER_W = BATCH // NW
ROWS_PER_GATHER = 2
IDX_PER_GATHER = ROWS_PER_GATHER * HIST
CHUNKS = ROWS_PER_W // ROWS_PER_GATHER

_mesh = plsc.VectorSubcoreMesh(core_axis_name="c", subcore_axis_name="s")


@functools.partial(
    pl.kernel,
    mesh=_mesh,
    out_type=jax.ShapeDtypeStruct((BATCH, DIM), jnp.float32),
    scratch_types=[
        pltpu.VMEM((CHUNKS, IDX_PER_GATHER), jnp.int32),
        pltpu.VMEM((IDX_PER_GATHER, DIM), jnp.float32),
        pltpu.VMEM((ROWS_PER_W, DIM), jnp.float32),
        pltpu.SemaphoreType.DMA,
    ],
)
def _cbow_sc(idx_hbm, table_hbm, out_hbm, idx_v, buf_v, out_v, sem):
    wid = lax.axis_index("s") * NC + lax.axis_index("c")

    pltpu.sync_copy(idx_hbm.at[wid], idx_v)

    zero = jnp.zeros((16,), jnp.float32)

    def chunk_body(c, _):
        pltpu.async_copy(table_hbm.at[idx_v.at[c]], buf_v, sem).wait()

        def accum_row(r):
            def h_body(h, accs):
                a0, a1, a2, a3 = accs
                hp = r * HIST + h
                a0 = a0 + buf_v[hp, pl.ds(0, 16)]
                a1 = a1 + buf_v[hp, pl.ds(16, 16)]
                a2 = a2 + buf_v[hp, pl.ds(32, 16)]
                a3 = a3 + buf_v[hp, pl.ds(48, 16)]
                return (a0, a1, a2, a3)

            a0, a1, a2, a3 = lax.fori_loop(
                0, HIST, h_body, (zero, zero, zero, zero))
            row = c * ROWS_PER_GATHER + r
            out_v[row, pl.ds(0, 16)] = a0
            out_v[row, pl.ds(16, 16)] = a1
            out_v[row, pl.ds(32, 16)] = a2
            out_v[row, pl.ds(48, 16)] = a3

        for r in range(ROWS_PER_GATHER):
            accum_row(r)
        return 0

    lax.fori_loop(0, CHUNKS, chunk_body, 0)

    pltpu.sync_copy(out_v, out_hbm.at[pl.ds(wid * ROWS_PER_W, ROWS_PER_W)])


def kernel(input_text, table):
    idx = input_text.reshape(NW, CHUNKS, IDX_PER_GATHER).astype(jnp.int32)
    return _cbow_sc(idx, table)


# speedup vs baseline: 2.1364x; 2.1364x over previous
import functools

import jax
import jax.numpy as jnp
from jax import lax
from jax.experimental import pallas as pl
from jax.experimental.pallas import tpu as pltpu
from jax.experimental.pallas import tpu_sc as plsc

VOCAB = 1000000
DIM = 64
BATCH = 16384
HIST = 50

NC = 2
NS = 16
NW = NC * NS
ROWS_PER_W = BATCH // NW
ROWS_PER_GATHER = 2
IDX_PER_GATHER = ROWS_PER_GATHER * HIST
CHUNKS = ROWS_PER_W // ROWS_PER_GATHER

_mesh = plsc.VectorSubcoreMesh(core_axis_name="c", subcore_axis_name="s")


@functools.partial(
    pl.kernel,
    mesh=_mesh,
    compiler_params=pltpu.CompilerParams(use_tc_tiling_on_sc=False),
    out_type=jax.ShapeDtypeStruct((BATCH, DIM), jnp.float32),
    scratch_types=[
        pltpu.VMEM((CHUNKS, IDX_PER_GATHER), jnp.int32),
        pltpu.VMEM((IDX_PER_GATHER, DIM), jnp.float32),
        pltpu.VMEM((ROWS_PER_W, DIM), jnp.float32),
        pltpu.SemaphoreType.DMA,
    ],
)
def _cbow_sc(idx_hbm, table_hbm, out_hbm, idx_v, buf_v, out_v, sem):
    wid = lax.axis_index("s") * NC + lax.axis_index("c")

    pltpu.sync_copy(idx_hbm.at[wid], idx_v)

    zero = jnp.zeros((16,), jnp.float32)

    def chunk_body(c, _):
        pltpu.async_copy(table_hbm.at[idx_v.at[c]], buf_v, sem).wait()

        def accum_row(r):
            def h_body(h, accs):
                a0, a1, a2, a3 = accs
                hp = r * HIST + h
                a0 = a0 + buf_v[hp, pl.ds(0, 16)]
                a1 = a1 + buf_v[hp, pl.ds(16, 16)]
                a2 = a2 + buf_v[hp, pl.ds(32, 16)]
                a3 = a3 + buf_v[hp, pl.ds(48, 16)]
                return (a0, a1, a2, a3)

            a0, a1, a2, a3 = lax.fori_loop(
                0, HIST, h_body, (zero, zero, zero, zero))
            row = c * ROWS_PER_GATHER + r
            out_v[row, pl.ds(0, 16)] = a0
            out_v[row, pl.ds(16, 16)] = a1
            out_v[row, pl.ds(32, 16)] = a2
            out_v[row, pl.ds(48, 16)] = a3

        for r in range(ROWS_PER_GATHER):
            accum_row(r)
        return 0

    lax.fori_loop(0, CHUNKS, chunk_body, 0)

    pltpu.sync_copy(out_v, out_hbm.at[pl.ds(wid * ROWS_PER_W, ROWS_PER_W)])


def kernel(input_text, table):
    idx = input_text.reshape(NW, CHUNKS, IDX_PER_GATHER).astype(jnp.int32)
    return _cbow_sc(idx, table)


# speedup vs baseline: 2.8114x; 1.3160x over previous
import functools

import jax
import jax.numpy as jnp
from jax import lax
from jax.experimental import pallas as pl
from jax.experimental.pallas import tpu as pltpu
from jax.experimental.pallas import tpu_sc as plsc

VOCAB = 1000000
DIM = 64
BATCH = 16384
HIST = 50

NC = 2
NS = 16
NW = NC * NS
ROWS_PER_W = BATCH // NW
ROWS_PER_GATHER = 2
IDX_PER_GATHER = ROWS_PER_GATHER * HIST
CHUNKS = ROWS_PER_W // ROWS_PER_GATHER
NBUF = 4
UNROLL = 5

_mesh = plsc.VectorSubcoreMesh(core_axis_name="c", subcore_axis_name="s")


@functools.partial(
    pl.kernel,
    mesh=_mesh,
    compiler_params=pltpu.CompilerParams(use_tc_tiling_on_sc=False),
    out_type=jax.ShapeDtypeStruct((BATCH, DIM), jnp.float32),
    scratch_types=[
        pltpu.VMEM((CHUNKS, IDX_PER_GATHER), jnp.int32),
        pltpu.VMEM((NBUF, IDX_PER_GATHER, DIM), jnp.float32),
        pltpu.VMEM((ROWS_PER_W, DIM), jnp.float32),
        pltpu.SemaphoreType.DMA((NBUF,)),
    ],
)
def _cbow_sc(idx_hbm, table_hbm, out_hbm, idx_v, bufs_v, out_v, sems):
    wid = lax.axis_index("s") * NC + lax.axis_index("c")

    pltpu.sync_copy(idx_hbm.at[wid], idx_v)

    zero = jnp.zeros((16,), jnp.float32)

    for b in range(NBUF):
        pltpu.async_copy(table_hbm.at[idx_v.at[b]], bufs_v.at[b], sems.at[b])

    def group_body(g, _):
        for b in range(NBUF):
            c = g * NBUF + b
            buf = bufs_v.at[b]
            pltpu.make_async_copy(
                table_hbm.at[idx_v.at[c]], buf, sems.at[b]).wait()

            for r in range(ROWS_PER_GATHER):
                def h_body(h, accs, r=r, buf=buf):
                    a0, a1, a2, a3 = accs
                    for u in range(UNROLL):
                        hp = r * HIST + h * UNROLL + u
                        a0 = a0 + buf[hp, pl.ds(0, 16)]
                        a1 = a1 + buf[hp, pl.ds(16, 16)]
                        a2 = a2 + buf[hp, pl.ds(32, 16)]
                        a3 = a3 + buf[hp, pl.ds(48, 16)]
                    return (a0, a1, a2, a3)

                a0, a1, a2, a3 = lax.fori_loop(
                    0, HIST // UNROLL, h_body, (zero, zero, zero, zero))
                row = c * ROWS_PER_GATHER + r
                out_v[row, pl.ds(0, 16)] = a0
                out_v[row, pl.ds(16, 16)] = a1
                out_v[row, pl.ds(32, 16)] = a2
                out_v[row, pl.ds(48, 16)] = a3

            nxt = c + NBUF
            @pl.when(nxt < CHUNKS)
            def _():
                pltpu.async_copy(
                    table_hbm.at[idx_v.at[nxt]], bufs_v.at[b], sems.at[b])
        return 0

    lax.fori_loop(0, CHUNKS // NBUF, group_body, 0)

    pltpu.sync_copy(out_v, out_hbm.at[pl.ds(wid * ROWS_PER_W, ROWS_PER_W)])


def kernel(input_text, table):
    idx = input_text.reshape(NW, CHUNKS, IDX_PER_GATHER).astype(jnp.int32)
    return _cbow_sc(idx, table)
